# baseline (speedup 1.0000x reference)
import functools

import jax
import jax.numpy as jnp
from jax import lax
from jax.experimental import pallas as pl
from jax.experimental.pallas import tpu as pltpu
from jax.experimental.pallas import tpu_sc as plsc

N_NODES = 10000
D = 128
DH = D // 2
N_EDGES = 320000

NC = 2
NS = 16

CHUNK = 128
CHUNKS_PER_W = 157
LOOP_CHUNKS = 156
EDGES_PER_W = CHUNK * CHUNKS_PER_W
E_PAD = EDGES_PER_W * NS
ROWS_PER_TILE = 640
N_PAD = ROWS_PER_TILE * NS

NBUF = 6

_mesh = plsc.VectorSubcoreMesh(core_axis_name="c", subcore_axis_name="s")


@functools.partial(
    pl.kernel,
    out_type=jax.ShapeDtypeStruct((NC, N_PAD, DH), jnp.float32),
    mesh=_mesh,
    scratch_types=[
        pltpu.VMEM((CHUNKS_PER_W, CHUNK), jnp.int32),
        pltpu.VMEM((CHUNKS_PER_W, CHUNK), jnp.int32),
        [pltpu.VMEM((CHUNK, DH), jnp.float32) for _ in range(NBUF)],
        pltpu.VMEM_SHARED((N_PAD, DH), jnp.float32),
        [pltpu.SemaphoreType.DMA for _ in range(NBUF)],
        [pltpu.SemaphoreType.DMA for _ in range(NBUF)],
    ],
    compiler_params=pltpu.CompilerParams(use_tc_tiling_on_sc=False),
)
def _sc_message_pass(x_hbm, src_hbm, dst_hbm, zeros_hbm, out_hbm,
                     src_v, dst_v, bufs, acc, gsem, ssem):
    c = lax.axis_index("c")
    s = lax.axis_index("s")

    pltpu.sync_copy(zeros_hbm, acc.at[pl.ds(s * ROWS_PER_TILE, ROWS_PER_TILE)])
    pltpu.sync_copy(src_hbm.at[s], src_v)
    pltpu.sync_copy(dst_hbm.at[s], dst_v)
    plsc.subcore_barrier()

    xc = x_hbm.at[c]

    def gather(j, b):
        return pltpu.async_copy(xc.at[src_v.at[j]], bufs[b], gsem[b])

    def gwait(j, b):
        pltpu.make_async_copy(xc.at[src_v.at[j]], bufs[b], gsem[b]).wait()

    def scatter(j, b):
        return pltpu.async_copy(bufs[b], acc.at[dst_v.at[j]], ssem[b],
                                add=True)

    def swait(j, b):
        pltpu.make_async_copy(bufs[b], acc.at[dst_v.at[j]], ssem[b]).wait()

    for j in range(4):
        gather(j, j)

    def body(j, b):
        gwait(j, b)
        scatter(j, b)
        b2 = (b + 4) % NBUF
        if isinstance(j, int) and j < 2:
            pass
        else:
            swait(j, b2)
        if isinstance(j, int) and j + 4 >= CHUNKS_PER_W:
            pass
        else:
            gather(j + 4, b2)

    for j in range(6):
        body(j, j % NBUF)

    @pl.loop(6, 150, step=NBUF)
    def _(j0):
        for b in range(NBUF):
            body(j0 + b, b)

    for j in range(150, CHUNKS_PER_W):
        body(j, j % NBUF)

    for j in (CHUNKS_PER_W - 2, CHUNKS_PER_W - 1):
        swait(0, j % NBUF)

    plsc.subcore_barrier()
    pltpu.sync_copy(acc.at[pl.ds(s * ROWS_PER_TILE, ROWS_PER_TILE)],
                    out_hbm.at[c, pl.ds(s * ROWS_PER_TILE, ROWS_PER_TILE)])


def _tc_body(relu, split_out, p_ref, g_ref, b_ref, w_ref, bias_ref, o_ref):
    h = jnp.concatenate([p_ref[0], p_ref[1]], axis=-1)
    mu = jnp.mean(h, axis=-1, keepdims=True)
    var = jnp.mean((h - mu) ** 2, axis=-1, keepdims=True)
    hn = (h - mu) * lax.rsqrt(var + 1e-5) * g_ref[...] + b_ref[...]
    y = lax.dot_general(hn, w_ref[...], (((1,), (1,)), ((), ())),
                        preferred_element_type=jnp.float32) + bias_ref[...]
    if relu:
        y = jnp.maximum(y, 0.0)
    if split_out:
        o_ref[0] = y[:, :DH]
        o_ref[1] = y[:, DH:]
    else:
        o_ref[...] = y


_TC_BLK = 400


def _tc_norm_linear(partials, g, b, w, bias, relu, split_out):
    body = functools.partial(_tc_body, relu, split_out)
    if split_out:
        out_shape = jax.ShapeDtypeStruct((NC, N_NODES, DH), jnp.float32)
        out_spec = pl.BlockSpec((NC, _TC_BLK, DH), lambda i: (0, i, 0))
    else:
        out_shape = jax.ShapeDtypeStruct((N_NODES, D), jnp.float32)
        out_spec = pl.BlockSpec((_TC_BLK, D), lambda i: (i, 0))
    return pl.pallas_call(
        body,
        grid=(N_NODES // _TC_BLK,),
        in_specs=[
            pl.BlockSpec((NC, _TC_BLK, DH), lambda i: (0, i, 0)),
            pl.BlockSpec((1, D), lambda i: (0, 0)),
            pl.BlockSpec((1, D), lambda i: (0, 0)),
            pl.BlockSpec((D, D), lambda i: (0, 0)),
            pl.BlockSpec((1, D), lambda i: (0, 0)),
        ],
        out_specs=out_spec,
        out_shape=out_shape,
    )(partials, g.reshape(1, D), b.reshape(1, D), w, bias.reshape(1, D))


def kernel(features, edge_index, W1, b1, ln1_g, ln1_b, W2, b2, ln2_g, ln2_b,
           W3, b3, ln3_g, ln3_b):
    src = edge_index[0].astype(jnp.int32)
    dst = edge_index[1].astype(jnp.int32)
    pad = E_PAD - N_EDGES
    src_p = jnp.concatenate([src, jnp.zeros((pad,), jnp.int32)])
    dst_p = jnp.concatenate([dst, jnp.full((pad,), N_NODES, jnp.int32)])
    src_p = src_p.reshape(NS, CHUNKS_PER_W, CHUNK)
    dst_p = dst_p.reshape(NS, CHUNKS_PER_W, CHUNK)
    zeros = jnp.zeros((ROWS_PER_TILE, DH), jnp.float32)

    x = jnp.stack([features[:, :DH], features[:, DH:]])
    for w, bias, g, b, relu in ((W1, b1, ln1_g, ln1_b, True),
                                (W2, b2, ln2_g, ln2_b, True),
                                (W3, b3, ln3_g, ln3_b, False)):
        partials = _sc_message_pass(x, src_p, dst_p, zeros)
        x = _tc_norm_linear(partials, g, b, w, bias, relu,
                            split_out=relu)
    return x


# speedup vs baseline: 1.0563x; 1.0563x over previous
import functools

import jax
import jax.numpy as jnp
from jax import lax
from jax.experimental import pallas as pl
from jax.experimental.pallas import tpu as pltpu
from jax.experimental.pallas import tpu_sc as plsc

N_NODES = 10000
D = 128
DH = D // 2
N_EDGES = 320000

NC = 2
NS = 16

CHUNK = 128
CHUNKS_PER_W = 157
LOOP_CHUNKS = 156
EDGES_PER_W = CHUNK * CHUNKS_PER_W
E_PAD = EDGES_PER_W * NS
ROWS_PER_TILE = 640
N_PAD = ROWS_PER_TILE * NS

NBUF = 6

_mesh = plsc.VectorSubcoreMesh(core_axis_name="c", subcore_axis_name="s")


@functools.partial(
    pl.kernel,
    out_type=jax.ShapeDtypeStruct((NC, N_PAD, DH), jnp.float32),
    mesh=_mesh,
    scratch_types=[
        pltpu.VMEM((CHUNKS_PER_W, CHUNK), jnp.int32),
        pltpu.VMEM((CHUNKS_PER_W, CHUNK), jnp.int32),
        [pltpu.VMEM((CHUNK, DH), jnp.float32) for _ in range(NBUF)],
        pltpu.VMEM_SHARED((N_PAD, DH), jnp.float32),
        [pltpu.SemaphoreType.DMA for _ in range(NBUF)],
        [pltpu.SemaphoreType.DMA for _ in range(NBUF)],
        pltpu.SemaphoreType.DMA,
    ],
    compiler_params=pltpu.CompilerParams(use_tc_tiling_on_sc=False),
)
def _sc_message_pass(x_hbm, src_hbm, dst_hbm, zeros_hbm, out_hbm,
                     src_v, dst_v, bufs, acc, gsem, ssem, zsem):
    c = lax.axis_index("c")
    s = lax.axis_index("s")

    rows = acc.at[pl.ds(s * ROWS_PER_TILE, ROWS_PER_TILE)]
    pltpu.async_copy(zeros_hbm, rows, zsem)
    pltpu.sync_copy(src_hbm.at[s], src_v)
    pltpu.sync_copy(dst_hbm.at[s], dst_v)

    xc = x_hbm.at[c]

    def gather(j, b):
        return pltpu.async_copy(xc.at[src_v.at[j]], bufs[b], gsem[b])

    def gwait(j, b):
        pltpu.make_async_copy(xc.at[src_v.at[j]], bufs[b], gsem[b]).wait()

    def scatter(j, b):
        return pltpu.async_copy(bufs[b], acc.at[dst_v.at[j]], ssem[b],
                                add=True)

    def swait(j, b):
        pltpu.make_async_copy(bufs[b], acc.at[dst_v.at[j]], ssem[b]).wait()

    for j in range(4):
        gather(j, j)
    pltpu.make_async_copy(zeros_hbm, rows, zsem).wait()
    plsc.subcore_barrier()

    @pl.loop(0, LOOP_CHUNKS, step=NBUF)
    def _(j0):
        for b in range(NBUF):
            j = j0 + b
            gwait(j, b)
            scatter(j, b)
            b2 = (b + 4) % NBUF

            @pl.when(j >= 2)
            def _():
                swait(j, b2)

            @pl.when(j + 4 < CHUNKS_PER_W)
            def _():
                gather(j + 4, b2)

    b_last = LOOP_CHUNKS % NBUF
    gwait(LOOP_CHUNKS, b_last)
    scatter(LOOP_CHUNKS, b_last)

    for j in range(LOOP_CHUNKS - 2, CHUNKS_PER_W):
        swait(0, j % NBUF)

    plsc.subcore_barrier()
    pltpu.sync_copy(acc.at[pl.ds(s * ROWS_PER_TILE, ROWS_PER_TILE)],
                    out_hbm.at[c, pl.ds(s * ROWS_PER_TILE, ROWS_PER_TILE)])


def _tc_body(relu, split_out, p_ref, g_ref, b_ref, w_ref, bias_ref, o_ref):
    h = jnp.concatenate([p_ref[0], p_ref[1]], axis=-1)
    mu = jnp.mean(h, axis=-1, keepdims=True)
    var = jnp.mean((h - mu) ** 2, axis=-1, keepdims=True)
    hn = (h - mu) * lax.rsqrt(var + 1e-5) * g_ref[...] + b_ref[...]
    y = lax.dot_general(hn, w_ref[...], (((1,), (1,)), ((), ())),
                        preferred_element_type=jnp.float32) + bias_ref[...]
    if relu:
        y = jnp.maximum(y, 0.0)
    if split_out:
        o_ref[0] = y[:, :DH]
        o_ref[1] = y[:, DH:]
    else:
        o_ref[...] = y


_TC_BLK = 1000


def _tc_norm_linear(partials, g, b, w, bias, relu, split_out):
    body = functools.partial(_tc_body, relu, split_out)
    if split_out:
        out_shape = jax.ShapeDtypeStruct((NC, N_NODES, DH), jnp.float32)
        out_spec = pl.BlockSpec((NC, _TC_BLK, DH), lambda i: (0, i, 0))
    else:
        out_shape = jax.ShapeDtypeStruct((N_NODES, D), jnp.float32)
        out_spec = pl.BlockSpec((_TC_BLK, D), lambda i: (i, 0))
    return pl.pallas_call(
        body,
        grid=(N_NODES // _TC_BLK,),
        in_specs=[
            pl.BlockSpec((NC, _TC_BLK, DH), lambda i: (0, i, 0)),
            pl.BlockSpec((1, D), lambda i: (0, 0)),
            pl.BlockSpec((1, D), lambda i: (0, 0)),
            pl.BlockSpec((D, D), lambda i: (0, 0)),
            pl.BlockSpec((1, D), lambda i: (0, 0)),
        ],
        out_specs=out_spec,
        out_shape=out_shape,
    )(partials, g.reshape(1, D), b.reshape(1, D), w, bias.reshape(1, D))


def kernel(features, edge_index, W1, b1, ln1_g, ln1_b, W2, b2, ln2_g, ln2_b,
           W3, b3, ln3_g, ln3_b):
    src = edge_index[0].astype(jnp.int32)
    dst = edge_index[1].astype(jnp.int32)
    pad = E_PAD - N_EDGES
    src_p = jnp.concatenate([src, jnp.zeros((pad,), jnp.int32)])
    dst_p = jnp.concatenate([dst, jnp.full((pad,), N_NODES, jnp.int32)])
    src_p = src_p.reshape(NS, CHUNKS_PER_W, CHUNK)
    dst_p = dst_p.reshape(NS, CHUNKS_PER_W, CHUNK)
    zeros = jnp.zeros((ROWS_PER_TILE, DH), jnp.float32)

    x = jnp.stack([features[:, :DH], features[:, DH:]])
    for w, bias, g, b, relu in ((W1, b1, ln1_g, ln1_b, True),
                                (W2, b2, ln2_g, ln2_b, True),
                                (W3, b3, ln3_g, ln3_b, False)):
        partials = _sc_message_pass(x, src_p, dst_p, zeros)
        x = _tc_norm_linear(partials, g, b, w, bias, relu,
                            split_out=relu)
    return x


# speedup vs baseline: 1.0819x; 1.0243x over previous
import functools

import jax
import jax.numpy as jnp
from jax import lax
from jax.experimental import pallas as pl
from jax.experimental.pallas import tpu as pltpu
from jax.experimental.pallas import tpu_sc as plsc

N_NODES = 10000
D = 128
DH = D // 2
N_EDGES = 320000

NC = 2
NS = 16

CHUNK = 128
CHUNKS_PER_W = 157
LOOP_CHUNKS = 156
EDGES_PER_W = CHUNK * CHUNKS_PER_W
E_PAD = EDGES_PER_W * NS
ROWS_PER_TILE = 640
N_PAD = ROWS_PER_TILE * NS

NBUF = 6

_mesh = plsc.VectorSubcoreMesh(core_axis_name="c", subcore_axis_name="s")


@functools.partial(
    pl.kernel,
    out_type=jax.ShapeDtypeStruct((NC, N_PAD, DH), jnp.float32),
    mesh=_mesh,
    scratch_types=[
        pltpu.VMEM((CHUNKS_PER_W, CHUNK), jnp.int32),
        pltpu.VMEM((CHUNKS_PER_W, CHUNK), jnp.int32),
        [pltpu.VMEM((CHUNK, DH), jnp.float32) for _ in range(NBUF)],
        pltpu.VMEM_SHARED((N_PAD, DH), jnp.float32),
        [pltpu.SemaphoreType.DMA for _ in range(NBUF)],
        [pltpu.SemaphoreType.DMA for _ in range(NBUF)],
        pltpu.SemaphoreType.DMA,
    ],
    compiler_params=pltpu.CompilerParams(use_tc_tiling_on_sc=False),
)
def _sc_message_pass(x_hbm, src_hbm, dst_hbm, zeros_hbm, out_hbm,
                     src_v, dst_v, bufs, acc, gsem, ssem, zsem):
    c = lax.axis_index("c")
    s = lax.axis_index("s")

    rows = acc.at[pl.ds(s * ROWS_PER_TILE, ROWS_PER_TILE)]
    pltpu.async_copy(zeros_hbm, rows, zsem)
    pltpu.sync_copy(src_hbm.at[s], src_v)
    pltpu.sync_copy(dst_hbm.at[s], dst_v)

    xc = x_hbm.at[c]

    def gather(j, b):
        return pltpu.async_copy(xc.at[src_v.at[j]], bufs[b], gsem[b])

    def gwait(j, b):
        pltpu.make_async_copy(xc.at[src_v.at[j]], bufs[b], gsem[b]).wait()

    def scatter(j, b):
        return pltpu.async_copy(bufs[b], acc.at[dst_v.at[j]], ssem[b],
                                add=True)

    def swait(j, b):
        pltpu.make_async_copy(bufs[b], acc.at[dst_v.at[j]], ssem[b]).wait()

    for j in range(4):
        gather(j, j)
    pltpu.make_async_copy(zeros_hbm, rows, zsem).wait()
    plsc.subcore_barrier()

    @pl.loop(0, LOOP_CHUNKS, step=NBUF)
    def _(j0):
        for b in range(NBUF):
            j = j0 + b
            gwait(j, b)
            scatter(j, b)
            b2 = (b + 4) % NBUF

            @pl.when(j >= 2)
            def _():
                swait(j, b2)

            @pl.when(j + 4 < CHUNKS_PER_W)
            def _():
                gather(j + 4, b2)

    b_last = LOOP_CHUNKS % NBUF
    gwait(LOOP_CHUNKS, b_last)
    scatter(LOOP_CHUNKS, b_last)

    for j in range(LOOP_CHUNKS - 2, CHUNKS_PER_W):
        swait(0, j % NBUF)

    plsc.subcore_barrier()
    pltpu.sync_copy(acc.at[pl.ds(s * ROWS_PER_TILE, ROWS_PER_TILE)],
                    out_hbm.at[c, pl.ds(s * ROWS_PER_TILE, ROWS_PER_TILE)])


def _tc_body(relu, split_out, p_ref, g_ref, b_ref, w_ref, bias_ref, o_ref):
    h = jnp.concatenate([p_ref[0], p_ref[1]], axis=-1)
    mu = jnp.mean(h, axis=-1, keepdims=True)
    var = jnp.mean((h - mu) ** 2, axis=-1, keepdims=True)
    hn = (h - mu) * lax.rsqrt(var + 1e-5) * g_ref[...] + b_ref[...]
    y = lax.dot_general(hn, w_ref[...], (((1,), (1,)), ((), ())),
                        preferred_element_type=jnp.float32) + bias_ref[...]
    if relu:
        y = jnp.maximum(y, 0.0)
    if split_out:
        o_ref[0] = y[:, :DH]
        o_ref[1] = y[:, DH:]
    else:
        o_ref[...] = y


_TC_BLK = 2000


def _tc_norm_linear(partials, g, b, w, bias, relu, split_out):
    body = functools.partial(_tc_body, relu, split_out)
    if split_out:
        out_shape = jax.ShapeDtypeStruct((NC, N_NODES, DH), jnp.float32)
        out_spec = pl.BlockSpec((NC, _TC_BLK, DH), lambda i: (0, i, 0))
    else:
        out_shape = jax.ShapeDtypeStruct((N_NODES, D), jnp.float32)
        out_spec = pl.BlockSpec((_TC_BLK, D), lambda i: (i, 0))
    return pl.pallas_call(
        body,
        grid=(N_NODES // _TC_BLK,),
        in_specs=[
            pl.BlockSpec((NC, _TC_BLK, DH), lambda i: (0, i, 0)),
            pl.BlockSpec((1, D), lambda i: (0, 0)),
            pl.BlockSpec((1, D), lambda i: (0, 0)),
            pl.BlockSpec((D, D), lambda i: (0, 0)),
            pl.BlockSpec((1, D), lambda i: (0, 0)),
        ],
        out_specs=out_spec,
        out_shape=out_shape,
    )(partials, g.reshape(1, D), b.reshape(1, D), w, bias.reshape(1, D))


def kernel(features, edge_index, W1, b1, ln1_g, ln1_b, W2, b2, ln2_g, ln2_b,
           W3, b3, ln3_g, ln3_b):
    src = edge_index[0].astype(jnp.int32)
    dst = edge_index[1].astype(jnp.int32)
    pad = E_PAD - N_EDGES
    src_p = jnp.concatenate([src, jnp.zeros((pad,), jnp.int32)])
    dst_p = jnp.concatenate([dst, jnp.full((pad,), N_NODES, jnp.int32)])
    src_p = src_p.reshape(NS, CHUNKS_PER_W, CHUNK)
    dst_p = dst_p.reshape(NS, CHUNKS_PER_W, CHUNK)
    zeros = jnp.zeros((ROWS_PER_TILE, DH), jnp.float32)

    x = jnp.stack([features[:, :DH], features[:, DH:]])
    for w, bias, g, b, relu in ((W1, b1, ln1_g, ln1_b, True),
                                (W2, b2, ln2_g, ln2_b, True),
                                (W3, b3, ln3_g, ln3_b, False)):
        partials = _sc_message_pass(x, src_p, dst_p, zeros)
        x = _tc_norm_linear(partials, g, b, w, bias, relu,
                            split_out=relu)
    return x
